# baseline (speedup 1.0000x reference)
import functools

import jax
import jax.numpy as jnp
from jax import lax
from jax.experimental import pallas as pl
from jax.experimental.pallas import tpu as pltpu
from jax.experimental.pallas import tpu_sc as plsc


def _sc_ctx_sum(xT, emb_table, n_workers=32, num_cores=2):
    ctx, batch = xT.shape
    _, d = emb_table.shape
    nb = batch // n_workers

    mesh = plsc.VectorSubcoreMesh(core_axis_name="c", subcore_axis_name="s")

    @functools.partial(
        pl.kernel,
        out_type=jax.ShapeDtypeStruct((batch, d), jnp.float32),
        mesh=mesh,
        scratch_types=[
            pltpu.VMEM((ctx, nb), jnp.int32),
            pltpu.VMEM((nb, d), jnp.float32),
            pltpu.SemaphoreType.DMA,
        ],
    )
    def sc_sum(xT_hbm, table_hbm, out_hbm, idx_v, acc_v, sem):
        wid = lax.axis_index("s") * num_cores + lax.axis_index("c")
        base = wid * nb
        pltpu.sync_copy(xT_hbm.at[:, pl.ds(base, nb)], idx_v)
        pltpu.async_copy(table_hbm.at[idx_v.at[0]], acc_v, sem).wait()
        adds = [
            pltpu.async_copy(table_hbm.at[idx_v.at[c]], acc_v, sem, add=True)
            for c in range(1, ctx)
        ]
        for cp in adds:
            cp.wait()
        pltpu.sync_copy(acc_v, out_hbm.at[pl.ds(base, nb)])

    return sc_sum(xT, emb_table)


def _tc_project(sums_bf16, W, b2, ctx, vt=256):
    batch, d = sums_bf16.shape
    vocab = W.shape[0]
    inv_ctx = 1.0 / ctx

    def body(s_ref, w_ref, b_ref, o_ref):
        w = (w_ref[...] * inv_ctx).astype(jnp.bfloat16)
        acc = lax.dot_general(
            s_ref[...], w, (((1,), (1,)), ((), ())),
            preferred_element_type=jnp.float32,
        )
        o_ref[...] = acc + b_ref[...]

    grid = (pl.cdiv(vocab, vt),)
    return pl.pallas_call(
        body,
        grid=grid,
        in_specs=[
            pl.BlockSpec((batch, d), lambda j: (0, 0)),
            pl.BlockSpec((vt, d), lambda j: (j, 0)),
            pl.BlockSpec((1, vt), lambda j: (0, j)),
        ],
        out_specs=pl.BlockSpec((batch, vt), lambda j: (0, j)),
        out_shape=jax.ShapeDtypeStruct((batch, vocab), jnp.float32),
    )(sums_bf16, W, b2)


def kernel(x, emb_table, W, b):
    ctx = x.shape[1]
    sums = _sc_ctx_sum(x.T, emb_table)
    return _tc_project(sums.astype(jnp.bfloat16), W, b.reshape(1, -1), ctx)


# speedup vs baseline: 1.0411x; 1.0411x over previous
import functools

import jax
import jax.numpy as jnp
from jax import lax
from jax.experimental import pallas as pl
from jax.experimental.pallas import tpu as pltpu
from jax.experimental.pallas import tpu_sc as plsc


def _sc_ctx_sum(xT, emb_table, n_workers=32, num_cores=2):
    ctx, batch = xT.shape
    _, d = emb_table.shape
    nb = batch // n_workers

    mesh = plsc.VectorSubcoreMesh(core_axis_name="c", subcore_axis_name="s")

    @functools.partial(
        pl.kernel,
        out_type=jax.ShapeDtypeStruct((batch, d), jnp.float32),
        mesh=mesh,
        scratch_types=[
            pltpu.VMEM((ctx, nb), jnp.int32),
            pltpu.VMEM((nb, d), jnp.float32),
            pltpu.SemaphoreType.DMA,
        ],
    )
    def sc_sum(xT_hbm, table_hbm, out_hbm, idx_v, acc_v, sem):
        wid = lax.axis_index("s") * num_cores + lax.axis_index("c")
        base = wid * nb
        pltpu.sync_copy(xT_hbm.at[:, pl.ds(base, nb)], idx_v)
        pltpu.async_copy(table_hbm.at[idx_v.at[0]], acc_v, sem).wait()
        adds = [
            pltpu.async_copy(table_hbm.at[idx_v.at[c]], acc_v, sem, add=True)
            for c in range(1, ctx)
        ]
        for cp in adds:
            cp.wait()
        pltpu.sync_copy(acc_v, out_hbm.at[pl.ds(base, nb)])

    return sc_sum(xT, emb_table)


def _tc_project(sums_bf16, W, b2, ctx, vt=512):
    batch, d = sums_bf16.shape
    vocab = W.shape[0]
    inv_ctx = 1.0 / ctx

    def body(s_ref, w_ref, b_ref, o_ref):
        w = (w_ref[...] * inv_ctx).astype(jnp.bfloat16)
        acc = lax.dot_general(
            s_ref[...], w, (((1,), (1,)), ((), ())),
            preferred_element_type=jnp.float32,
        )
        o_ref[...] = acc + b_ref[...]

    grid = (pl.cdiv(vocab, vt),)
    return pl.pallas_call(
        body,
        grid=grid,
        in_specs=[
            pl.BlockSpec((batch, d), lambda j: (0, 0)),
            pl.BlockSpec((vt, d), lambda j: (j, 0)),
            pl.BlockSpec((1, vt), lambda j: (0, j)),
        ],
        out_specs=pl.BlockSpec((batch, vt), lambda j: (0, j)),
        out_shape=jax.ShapeDtypeStruct((batch, vocab), jnp.float32),
    )(sums_bf16, W, b2)


def kernel(x, emb_table, W, b):
    ctx = x.shape[1]
    sums = _sc_ctx_sum(x.T, emb_table)
    return _tc_project(sums.astype(jnp.bfloat16), W, b.reshape(1, -1), ctx)


# speedup vs baseline: 1.0434x; 1.0022x over previous
import functools

import jax
import jax.numpy as jnp
from jax import lax
from jax.experimental import pallas as pl
from jax.experimental.pallas import tpu as pltpu
from jax.experimental.pallas import tpu_sc as plsc


def _sc_ctx_sum(xT, emb_table, n_workers=32, num_cores=2):
    ctx, batch = xT.shape
    _, d = emb_table.shape
    nb = batch // n_workers

    mesh = plsc.VectorSubcoreMesh(core_axis_name="c", subcore_axis_name="s")

    @functools.partial(
        pl.kernel,
        out_type=jax.ShapeDtypeStruct((batch, d), jnp.float32),
        mesh=mesh,
        scratch_types=[
            pltpu.VMEM((ctx, nb), jnp.int32),
            pltpu.VMEM((nb, d), jnp.float32),
            pltpu.SemaphoreType.DMA,
        ],
    )
    def sc_sum(xT_hbm, table_hbm, out_hbm, idx_v, acc_v, sem):
        wid = lax.axis_index("s") * num_cores + lax.axis_index("c")
        base = wid * nb
        pltpu.sync_copy(xT_hbm.at[:, pl.ds(base, nb)], idx_v)
        pltpu.async_copy(table_hbm.at[idx_v.at[0]], acc_v, sem).wait()
        adds = [
            pltpu.async_copy(table_hbm.at[idx_v.at[c]], acc_v, sem, add=True)
            for c in range(1, ctx)
        ]
        for cp in adds:
            cp.wait()
        pltpu.sync_copy(acc_v, out_hbm.at[pl.ds(base, nb)])

    return sc_sum(xT, emb_table)


def _tc_project(sums_bf16, W, b2, ctx, vt=512, rows_per_dma=512):
    batch, d = sums_bf16.shape
    vocab = W.shape[0]
    inv_ctx = 1.0 / ctx
    nv = pl.cdiv(vocab, vt)
    tail = ((vocab - (nv - 1) * vt + 127) // 128) * 128
    nr = batch // rows_per_dma
    p_last = (nv - 1) % 2

    def body(s_ref, w_ref, b_ref, o_hbm, o_v0, o_v1, sem0, sem1):
        j = pl.program_id(0)
        bufs = ((o_v0, sem0), (o_v1, sem1))

        def fire(o_v, sem, width):
            for r in range(nr):
                rows = pl.ds(r * rows_per_dma, rows_per_dma)
                pltpu.async_copy(
                    o_v.at[rows, pl.ds(0, width)],
                    o_hbm.at[rows, pl.ds(j * vt, width)],
                    sem,
                )

        def drain(o_v, sem, width):
            for r in range(nr):
                rows = pl.ds(r * rows_per_dma, rows_per_dma)
                pltpu.make_async_copy(
                    o_v.at[rows, pl.ds(0, width)],
                    o_hbm.at[rows, pl.ds(0, width)],
                    sem,
                ).wait()

        for p in range(2):
            o_v, sem = bufs[p]

            @pl.when(j % 2 == p)
            def _(o_v=o_v, sem=sem):
                @pl.when(j >= 2)
                def _():
                    drain(o_v, sem, vt)

                w = (w_ref[...] * inv_ctx).astype(jnp.bfloat16)
                o_v[...] = lax.dot_general(
                    s_ref[...], w, (((1,), (1,)), ((), ())),
                    preferred_element_type=jnp.float32,
                ) + b_ref[...]

                @pl.when(j < nv - 1)
                def _():
                    fire(o_v, sem, vt)

                @pl.when(j == nv - 1)
                def _():
                    fire(o_v, sem, tail)

        @pl.when(j == nv - 1)
        def _():
            drain(*bufs[p_last], tail)
            if nv >= 2:
                drain(*bufs[1 - p_last], vt)

    return pl.pallas_call(
        body,
        grid=(nv,),
        in_specs=[
            pl.BlockSpec((batch, d), lambda j: (0, 0)),
            pl.BlockSpec((vt, d), lambda j: (j, 0)),
            pl.BlockSpec((1, vt), lambda j: (0, j)),
        ],
        out_specs=pl.BlockSpec(memory_space=pltpu.HBM),
        out_shape=jax.ShapeDtypeStruct((batch, vocab), jnp.float32),
        scratch_shapes=[
            pltpu.VMEM((batch, vt), jnp.float32),
            pltpu.VMEM((batch, vt), jnp.float32),
            pltpu.SemaphoreType.DMA,
            pltpu.SemaphoreType.DMA,
        ],
        compiler_params=pltpu.CompilerParams(
            vmem_limit_bytes=60 * 1024 * 1024,
        ),
    )(sums_bf16, W, b2)


def kernel(x, emb_table, W, b):
    ctx = x.shape[1]
    sums = _sc_ctx_sum(x.T, emb_table)
    return _tc_project(sums.astype(jnp.bfloat16), W, b.reshape(1, -1), ctx)


# speedup vs baseline: 3.4313x; 3.2887x over previous
import functools

import jax
import jax.numpy as jnp
from jax import lax
from jax.experimental import pallas as pl
from jax.experimental.pallas import tpu as pltpu
from jax.experimental.pallas import tpu_sc as plsc


def _sc_ctx_sum(xT, emb_table, n_workers=32, num_cores=2):
    ctx, batch = xT.shape
    _, d = emb_table.shape
    nb = batch // n_workers

    mesh = plsc.VectorSubcoreMesh(core_axis_name="c", subcore_axis_name="s")

    @functools.partial(
        pl.kernel,
        out_type=jax.ShapeDtypeStruct((batch, d), jnp.float32),
        mesh=mesh,
        scratch_types=[
            pltpu.VMEM((ctx, nb), jnp.int32),
            pltpu.VMEM((nb, d), jnp.float32),
            pltpu.SemaphoreType.DMA,
        ],
    )
    def sc_sum(xT_hbm, table_hbm, out_hbm, idx_v, acc_v, sem):
        wid = lax.axis_index("s") * num_cores + lax.axis_index("c")
        base = wid * nb
        pltpu.sync_copy(xT_hbm.at[:, pl.ds(base, nb)], idx_v)
        pltpu.async_copy(table_hbm.at[idx_v.at[0]], acc_v, sem).wait()
        adds = [
            pltpu.async_copy(table_hbm.at[idx_v.at[c]], acc_v, sem, add=True)
            for c in range(1, ctx)
        ]
        for cp in adds:
            cp.wait()
        pltpu.sync_copy(acc_v, out_hbm.at[pl.ds(base, nb)])

    return sc_sum(xT, emb_table)


def _tc_project_t(sums_bf16, W, bcol, ctx, vt=512):
    batch, d = sums_bf16.shape
    vocab = W.shape[0]
    inv_ctx = 1.0 / ctx

    def body(s_ref, w_ref, b_ref, o_ref):
        w = (w_ref[...] * inv_ctx).astype(jnp.bfloat16)
        o_ref[...] = lax.dot_general(
            w, s_ref[...], (((1,), (1,)), ((), ())),
            preferred_element_type=jnp.float32,
        ) + b_ref[...]

    return pl.pallas_call(
        body,
        grid=(pl.cdiv(vocab, vt),),
        in_specs=[
            pl.BlockSpec((batch, d), lambda j: (0, 0)),
            pl.BlockSpec((vt, d), lambda j: (j, 0)),
            pl.BlockSpec((vt, 1), lambda j: (j, 0)),
        ],
        out_specs=pl.BlockSpec((vt, batch), lambda j: (j, 0)),
        out_shape=jax.ShapeDtypeStruct((vocab, batch), jnp.float32),
    )(sums_bf16, W, bcol)


def kernel(x, emb_table, W, b):
    ctx = x.shape[1]
    sums = _sc_ctx_sum(x.T, emb_table)
    logits_t = _tc_project_t(
        sums.astype(jnp.bfloat16), W, b.reshape(-1, 1), ctx)
    return logits_t.T


# speedup vs baseline: 3.7062x; 1.0801x over previous
import functools

import jax
import jax.numpy as jnp
from jax import lax
from jax.experimental import pallas as pl
from jax.experimental.pallas import tpu as pltpu
from jax.experimental.pallas import tpu_sc as plsc


def _sc_ctx_sum(xT, emb_table, n_workers=32, num_cores=2):
    ctx, batch = xT.shape
    _, d = emb_table.shape
    nb = batch // n_workers

    mesh = plsc.VectorSubcoreMesh(core_axis_name="c", subcore_axis_name="s")

    @functools.partial(
        pl.kernel,
        out_type=jax.ShapeDtypeStruct((batch, d), jnp.float32),
        mesh=mesh,
        scratch_types=[
            pltpu.VMEM((ctx, nb), jnp.int32),
            pltpu.VMEM((nb, d), jnp.float32),
            pltpu.SemaphoreType.DMA,
        ],
    )
    def sc_sum(xT_hbm, table_hbm, out_hbm, idx_v, acc_v, sem):
        wid = lax.axis_index("s") * num_cores + lax.axis_index("c")
        base = wid * nb
        pltpu.sync_copy(xT_hbm.at[:, pl.ds(base, nb)], idx_v)
        pltpu.async_copy(table_hbm.at[idx_v.at[0]], acc_v, sem).wait()
        adds = [
            pltpu.async_copy(table_hbm.at[idx_v.at[c]], acc_v, sem, add=True)
            for c in range(1, ctx)
        ]
        for cp in adds:
            cp.wait()
        pltpu.sync_copy(acc_v, out_hbm.at[pl.ds(base, nb)])

    return sc_sum(xT, emb_table)


def _tc_project_t(sums_bf16, W, bcol, ctx, vt=512):
    batch, d = sums_bf16.shape
    vocab = W.shape[0]
    inv_ctx = 1.0 / ctx

    def body(s_ref, w_ref, b_ref, o_ref):
        w = (w_ref[...] * inv_ctx).astype(jnp.bfloat16)
        o_ref[...] = lax.dot_general(
            w, s_ref[...], (((1,), (1,)), ((), ())),
            preferred_element_type=jnp.float32,
        ) + b_ref[...].T

    return pl.pallas_call(
        body,
        grid=(pl.cdiv(vocab, vt),),
        in_specs=[
            pl.BlockSpec((batch, d), lambda j: (0, 0)),
            pl.BlockSpec((vt, d), lambda j: (j, 0)),
            pl.BlockSpec((1, vt), lambda j: (0, j)),
        ],
        out_specs=pl.BlockSpec((vt, batch), lambda j: (j, 0)),
        out_shape=jax.ShapeDtypeStruct((vocab, batch), jnp.float32),
    )(sums_bf16, W, bcol)


def kernel(x, emb_table, W, b):
    ctx = x.shape[1]
    sums = _sc_ctx_sum(x.T, emb_table)
    logits_t = _tc_project_t(
        sums.astype(jnp.bfloat16), W, b.reshape(1, -1), ctx)
    return logits_t.T


# speedup vs baseline: 3.7630x; 1.0153x over previous
import functools

import jax
import jax.numpy as jnp
from jax import lax
from jax.experimental import pallas as pl
from jax.experimental.pallas import tpu as pltpu
from jax.experimental.pallas import tpu_sc as plsc


def _sc_ctx_sum(xT, emb_table, n_workers=32, num_cores=2):
    ctx, batch = xT.shape
    _, d = emb_table.shape
    nb = batch // n_workers

    mesh = plsc.VectorSubcoreMesh(core_axis_name="c", subcore_axis_name="s")

    @functools.partial(
        pl.kernel,
        out_type=jax.ShapeDtypeStruct((batch, d), jnp.float32),
        mesh=mesh,
        scratch_types=[
            pltpu.VMEM((ctx, nb), jnp.int32),
            pltpu.VMEM((nb, d), jnp.float32),
            pltpu.SemaphoreType.DMA,
        ],
    )
    def sc_sum(xT_hbm, table_hbm, out_hbm, idx_v, acc_v, sem):
        wid = lax.axis_index("s") * num_cores + lax.axis_index("c")
        base = wid * nb
        pltpu.sync_copy(xT_hbm.at[:, pl.ds(base, nb)], idx_v)
        pltpu.async_copy(table_hbm.at[idx_v.at[0]], acc_v, sem).wait()
        adds = [
            pltpu.async_copy(table_hbm.at[idx_v.at[c]], acc_v, sem, add=True)
            for c in range(1, ctx)
        ]
        for cp in adds:
            cp.wait()
        pltpu.sync_copy(acc_v, out_hbm.at[pl.ds(base, nb)])

    return sc_sum(xT, emb_table)


def _tc_project_t(sums_bf16, W, bcol, ctx, vt=1024):
    batch, d = sums_bf16.shape
    vocab = W.shape[0]
    inv_ctx = 1.0 / ctx

    def body(s_ref, w_ref, b_ref, o_ref):
        w = (w_ref[...] * inv_ctx).astype(jnp.bfloat16)
        o_ref[...] = lax.dot_general(
            w, s_ref[...], (((1,), (1,)), ((), ())),
            preferred_element_type=jnp.float32,
        ) + b_ref[...].T

    return pl.pallas_call(
        body,
        grid=(pl.cdiv(vocab, vt),),
        in_specs=[
            pl.BlockSpec((batch, d), lambda j: (0, 0)),
            pl.BlockSpec((vt, d), lambda j: (j, 0)),
            pl.BlockSpec((1, vt), lambda j: (0, j)),
        ],
        out_specs=pl.BlockSpec((vt, batch), lambda j: (j, 0)),
        out_shape=jax.ShapeDtypeStruct((vocab, batch), jnp.float32),
    )(sums_bf16, W, bcol)


def kernel(x, emb_table, W, b):
    ctx = x.shape[1]
    sums = _sc_ctx_sum(x.T, emb_table)
    logits_t = _tc_project_t(
        sums.astype(jnp.bfloat16), W, b.reshape(1, -1), ctx)
    return logits_t.T


# speedup vs baseline: 3.7978x; 1.0092x over previous
import functools

import jax
import jax.numpy as jnp
from jax import lax
from jax.experimental import pallas as pl
from jax.experimental.pallas import tpu as pltpu
from jax.experimental.pallas import tpu_sc as plsc


def _sc_ctx_sum(xT, emb_table, n_workers=32, num_cores=2):
    ctx, batch = xT.shape
    _, d = emb_table.shape
    nb = batch // n_workers

    mesh = plsc.VectorSubcoreMesh(core_axis_name="c", subcore_axis_name="s")

    @functools.partial(
        pl.kernel,
        out_type=jax.ShapeDtypeStruct((batch, d), jnp.float32),
        mesh=mesh,
        scratch_types=[
            pltpu.VMEM((ctx, nb), jnp.int32),
            pltpu.VMEM((nb, d), jnp.float32),
            pltpu.SemaphoreType.DMA,
        ],
    )
    def sc_sum(xT_hbm, table_hbm, out_hbm, idx_v, acc_v, sem):
        wid = lax.axis_index("s") * num_cores + lax.axis_index("c")
        base = wid * nb
        pltpu.sync_copy(xT_hbm.at[:, pl.ds(base, nb)], idx_v)
        pltpu.async_copy(table_hbm.at[idx_v.at[0]], acc_v, sem).wait()
        adds = [
            pltpu.async_copy(table_hbm.at[idx_v.at[c]], acc_v, sem, add=True)
            for c in range(1, ctx)
        ]
        for cp in adds:
            cp.wait()
        pltpu.sync_copy(acc_v, out_hbm.at[pl.ds(base, nb)])

    return sc_sum(xT, emb_table)


def _tc_project_t(sums_t_bf16, W, bcol, ctx, vt=1024):
    d, batch = sums_t_bf16.shape
    vocab = W.shape[0]
    inv_ctx = 1.0 / ctx

    def body(s_ref, w_ref, b_ref, o_ref):
        w = (w_ref[...] * inv_ctx).astype(jnp.bfloat16)
        o_ref[...] = lax.dot_general(
            w, s_ref[...], (((1,), (0,)), ((), ())),
            preferred_element_type=jnp.float32,
        ) + b_ref[...].T

    return pl.pallas_call(
        body,
        grid=(pl.cdiv(vocab, vt),),
        in_specs=[
            pl.BlockSpec((d, batch), lambda j: (0, 0)),
            pl.BlockSpec((vt, d), lambda j: (j, 0)),
            pl.BlockSpec((1, vt), lambda j: (0, j)),
        ],
        out_specs=pl.BlockSpec((vt, batch), lambda j: (j, 0)),
        out_shape=jax.ShapeDtypeStruct((vocab, batch), jnp.float32),
        compiler_params=pltpu.CompilerParams(
            vmem_limit_bytes=100 * 1024 * 1024,
        ),
    )(sums_t_bf16, W, bcol)


def kernel(x, emb_table, W, b):
    ctx = x.shape[1]
    sums = _sc_ctx_sum(x.T, emb_table)
    logits_t = _tc_project_t(
        sums.T.astype(jnp.bfloat16), W, b.reshape(1, -1), ctx)
    return logits_t.T
